# baseline (speedup 1.0000x reference)
import jax
import jax.numpy as jnp
from jax import lax
from jax.experimental import pallas as pl
from jax.experimental.pallas import tpu as pltpu
from jax.experimental.pallas import tpu_sc as plsc

B = 1024
L = 77
D = 512
R = B * L
NC = 2
NS = 16
NW = NC * NS
RW = R // NW
C = 88
NCHUNK = RW // C
LT = 616


def _body(x_hbm, tok_hbm, pos_hbm, out_hbm, idx_v, buf, pbuf, sem):
    wid = lax.axis_index("s") * NC + lax.axis_index("c")
    base = wid * RW
    pltpu.sync_copy(x_hbm.at[pl.ds(base, RW)], idx_v)

    def per_chunk(c, _):
        off = c * C
        phase = lax.rem(base + off, LT)
        gather = pltpu.async_copy(tok_hbm.at[idx_v.at[pl.ds(off, C)]], buf, sem)
        pltpu.sync_copy(pos_hbm.at[pl.ds(phase, C)], pbuf)
        gather.wait()

        def per_row(r, _):
            for g in range(D // 16):
                sl = pl.ds(g * 16, 16)
                buf[r, sl] += pbuf[r, sl]
            return 0

        lax.fori_loop(0, C, per_row, 0)
        pltpu.sync_copy(buf, out_hbm.at[pl.ds(base + off, C)])
        return 0

    lax.fori_loop(0, NCHUNK, per_chunk, 0)


@jax.jit
def _cliphead(xf, token_embedding, pos616):
    kfn = pl.kernel(
        _body,
        out_type=jax.ShapeDtypeStruct((R, D), jnp.float32),
        mesh=plsc.VectorSubcoreMesh(core_axis_name="c", subcore_axis_name="s"),
        scratch_types=[
            pltpu.VMEM((RW,), jnp.int32),
            pltpu.VMEM((C, D), jnp.float32),
            pltpu.VMEM((C, D), jnp.float32),
            pltpu.SemaphoreType.DMA,
        ],
    )
    return kfn(xf, token_embedding, pos616)


def kernel(x, token_embedding, position_embedding):
    xf = x.astype(jnp.int32).reshape(R)
    pos616 = jnp.tile(position_embedding, (LT // L, 1))
    out = _cliphead(xf, token_embedding, pos616)
    return out.reshape(B, L, D)
